# Initial kernel scaffold; baseline (speedup 1.0000x reference)
import jax
import jax.numpy as jnp
from jax.experimental import pallas as pl


def kernel(v_labels, vectors):
    raise NotImplementedError("write your pallas kernel here")


# speedup vs baseline: 1.9720x; 1.9720x over previous
import jax
import jax.numpy as jnp
from jax.experimental import pallas as pl
from jax.experimental.pallas import tpu as pltpu

Q = 1024
D = 384
BK = 2000


def _knn_body(q_ref, v_ref, idx_out, val_out, qn_ref, best_ref, bidx_ref):
    j = pl.program_id(0)
    nb = pl.num_programs(0)

    @pl.when(j == 0)
    def _init():
        q = q_ref[...]
        qnorm = jnp.sqrt(jnp.sum(q * q, axis=1, keepdims=True))
        qn_ref[...] = q / jnp.maximum(qnorm, 1e-12)
        best_ref[...] = jnp.full((1, Q), -jnp.inf, jnp.float32)
        bidx_ref[...] = jnp.zeros((1, Q), jnp.int32)

    v = v_ref[...]
    vnorm = jnp.sqrt(jnp.sum(v * v, axis=1, keepdims=True))
    vn = v / jnp.maximum(vnorm, 1e-12)
    sims = jax.lax.dot_general(
        vn.astype(jnp.bfloat16), qn_ref[...].astype(jnp.bfloat16),
        (((1,), (1,)), ((), ())),
        preferred_element_type=jnp.float32)
    bmax = jnp.max(sims, axis=0)[None, :]
    barg = jnp.argmax(sims, axis=0)[None, :].astype(jnp.int32)
    upd = bmax > best_ref[...]
    bidx_ref[...] = jnp.where(upd, barg + j * BK, bidx_ref[...])
    best_ref[...] = jnp.where(upd, bmax, best_ref[...])

    @pl.when(j == nb - 1)
    def _fin():
        idx_out[...] = bidx_ref[...]
        val_out[...] = best_ref[...]


def kernel(v_labels, vectors):
    k = vectors.shape[0]
    nb = k // BK
    idx, val = pl.pallas_call(
        _knn_body,
        grid=(nb,),
        in_specs=[
            pl.BlockSpec((Q, D), lambda j: (0, 0)),
            pl.BlockSpec((BK, D), lambda j: (j, 0)),
        ],
        out_specs=[
            pl.BlockSpec((1, Q), lambda j: (0, 0)),
            pl.BlockSpec((1, Q), lambda j: (0, 0)),
        ],
        out_shape=[
            jax.ShapeDtypeStruct((1, Q), jnp.int32),
            jax.ShapeDtypeStruct((1, Q), jnp.float32),
        ],
        scratch_shapes=[
            pltpu.VMEM((Q, D), jnp.float32),
            pltpu.VMEM((1, Q), jnp.float32),
            pltpu.VMEM((1, Q), jnp.int32),
        ],
    )(v_labels, vectors)
    return idx.reshape(Q), val.reshape(Q)
